# Initial kernel scaffold; baseline (speedup 1.0000x reference)
import jax
import jax.numpy as jnp
from jax.experimental import pallas as pl


def kernel(x, edge_index, pos_edge_index, neg_edge_index, W1, b1, W2, b2):
    raise NotImplementedError("write your pallas kernel here")


# speedup vs baseline: 2.8104x; 2.8104x over previous
import functools

import jax
import jax.numpy as jnp
from jax import lax
from jax.experimental import pallas as pl
from jax.experimental.pallas import tpu as pltpu

_N = 10000
_E = 320000
_D = 128
_ROWS = 1000


def _prep_body(degp_ref, dinv_ref):
    dinv = lax.rsqrt(degp_ref[0] + degp_ref[1] + 1.0)
    dinv_ref[...] = jnp.broadcast_to(dinv[:, None], (_N, _D))


def _prep(deg_parts):
    return pl.pallas_call(
        _prep_body,
        out_shape=jax.ShapeDtypeStruct((_N, _D), jnp.float32),
    )(deg_parts)


def _enc1_body(x_ref, w1_ref, dinv_ref, h1p_ref):
    h = jnp.dot(x_ref[...], w1_ref[...], preferred_element_type=jnp.float32)
    h1p_ref[...] = h * dinv_ref[...]


def _enc1(x, w1, dinv):
    grid = _N // _ROWS
    return pl.pallas_call(
        _enc1_body,
        grid=(grid,),
        in_specs=[
            pl.BlockSpec((_ROWS, _D), lambda i: (i, 0)),
            pl.BlockSpec((_D, _D), lambda i: (0, 0)),
            pl.BlockSpec((_ROWS, _D), lambda i: (i, 0)),
        ],
        out_specs=pl.BlockSpec((_ROWS, _D), lambda i: (i, 0)),
        out_shape=jax.ShapeDtypeStruct((_N, _D), jnp.float32),
    )(x, w1, dinv)


def _enc2_body(s1_ref, h1p_ref, dinv_ref, b1_ref, w2_ref, h2p_ref):
    dinv = dinv_ref[...]
    t = (s1_ref[0] + s1_ref[1] + h1p_ref[...]) * dinv + b1_ref[...]
    o1 = jnp.maximum(t, 0.0)
    h2 = jnp.dot(o1, w2_ref[...], preferred_element_type=jnp.float32)
    h2p_ref[...] = h2 * dinv


def _enc2(s1, h1p, dinv, b1, w2):
    grid = _N // _ROWS
    return pl.pallas_call(
        _enc2_body,
        grid=(grid,),
        in_specs=[
            pl.BlockSpec((2, _ROWS, _D), lambda i: (0, i, 0)),
            pl.BlockSpec((_ROWS, _D), lambda i: (i, 0)),
            pl.BlockSpec((_ROWS, _D), lambda i: (i, 0)),
            pl.BlockSpec((1, _D), lambda i: (0, 0)),
            pl.BlockSpec((_D, _D), lambda i: (0, 0)),
        ],
        out_specs=pl.BlockSpec((_ROWS, _D), lambda i: (i, 0)),
        out_shape=jax.ShapeDtypeStruct((_N, _D), jnp.float32),
    )(s1, h1p, dinv, b1, w2)


def _final_body(s2_ref, h2p_ref, dinv_ref, b2_ref, z_ref):
    z_ref[...] = (s2_ref[0] + s2_ref[1] + h2p_ref[...]) * dinv_ref[...] + b2_ref[...]


def _final(s2, h2p, dinv, b2):
    grid = _N // _ROWS
    return pl.pallas_call(
        _final_body,
        grid=(grid,),
        in_specs=[
            pl.BlockSpec((2, _ROWS, _D), lambda i: (0, i, 0)),
            pl.BlockSpec((_ROWS, _D), lambda i: (i, 0)),
            pl.BlockSpec((_ROWS, _D), lambda i: (i, 0)),
            pl.BlockSpec((1, _D), lambda i: (0, 0)),
        ],
        out_specs=pl.BlockSpec((_ROWS, _D), lambda i: (i, 0)),
        out_shape=jax.ShapeDtypeStruct((_N, _D), jnp.float32),
    )(s2, h2p, dinv, b2)


def _deg_parts(edge_index):
    dst = edge_index[1]
    half = _E // 2
    d0 = jnp.zeros((_N,), jnp.float32).at[dst[:half]].add(1.0)
    d1 = jnp.zeros((_N,), jnp.float32).at[dst[half:]].add(1.0)
    return jnp.stack([d0, d1])


def _scatter_parts(edge_index, h):
    src, dst = edge_index[0], edge_index[1]
    half = _E // 2
    s0 = jnp.zeros((_N, _D), jnp.float32).at[dst[:half]].add(h[src[:half]])
    s1 = jnp.zeros((_N, _D), jnp.float32).at[dst[half:]].add(h[src[half:]])
    return jnp.stack([s0, s1])


def _decode(ei, z):
    return (z[ei[0]] * z[ei[1]]).sum(axis=-1)


def kernel(x, edge_index, pos_edge_index, neg_edge_index, W1, b1, W2, b2):
    deg_parts = _deg_parts(edge_index)
    dinv = _prep(deg_parts)
    h1p = _enc1(x, W1, dinv)
    s1 = _scatter_parts(edge_index, h1p)
    h2p = _enc2(s1, h1p, dinv, b1.reshape(1, _D), W2)
    s2 = _scatter_parts(edge_index, h2p)
    z = _final(s2, h2p, dinv, b2.reshape(1, _D))
    ei = jnp.concatenate([pos_edge_index, neg_edge_index], axis=-1)
    return _decode(ei, z)


# speedup vs baseline: 8.3898x; 2.9853x over previous
import functools

import jax
import jax.numpy as jnp
from jax import lax
from jax.experimental import pallas as pl
from jax.experimental.pallas import tpu as pltpu
from jax.experimental.pallas import tpu_sc as plsc

_N = 10000
_E = 320000
_D = 128
_ROWS = 1000

_NP = 10240
_RPT = _NP // 16
_NW = 32
_EPT = _E // _NW
_CH = 80
_NCH = _EPT // _CH

_SC_MESH = dict(core_axis_name="c", subcore_axis_name="s")


def _prep_body(degp_ref, dinv_ref):
    dinv = lax.rsqrt(degp_ref[0] + degp_ref[1] + 1.0)
    dinv_ref[...] = jnp.broadcast_to(dinv[:, None], (_N, _D))


def _prep(deg_parts):
    return pl.pallas_call(
        _prep_body,
        out_shape=jax.ShapeDtypeStruct((_N, _D), jnp.float32),
    )(deg_parts)


def _enc1_body(x_ref, w1_ref, dinv_ref, h1p_ref):
    h = jnp.dot(x_ref[...], w1_ref[...], preferred_element_type=jnp.float32)
    h1p_ref[...] = h * dinv_ref[...]


def _enc1(x, w1, dinv):
    grid = _N // _ROWS
    return pl.pallas_call(
        _enc1_body,
        grid=(grid,),
        in_specs=[
            pl.BlockSpec((_ROWS, _D), lambda i: (i, 0)),
            pl.BlockSpec((_D, _D), lambda i: (0, 0)),
            pl.BlockSpec((_ROWS, _D), lambda i: (i, 0)),
        ],
        out_specs=pl.BlockSpec((_ROWS, _D), lambda i: (i, 0)),
        out_shape=jax.ShapeDtypeStruct((_N, _D), jnp.float32),
    )(x, w1, dinv)


def _enc2_body(s1_ref, h1p_ref, dinv_ref, b1_ref, w2_ref, h2p_ref):
    dinv = dinv_ref[...]
    t = (s1_ref[0] + s1_ref[1] + h1p_ref[...]) * dinv + b1_ref[...]
    o1 = jnp.maximum(t, 0.0)
    h2 = jnp.dot(o1, w2_ref[...], preferred_element_type=jnp.float32)
    h2p_ref[...] = h2 * dinv


def _enc2(s1, h1p, dinv, b1, w2):
    grid = _N // _ROWS
    return pl.pallas_call(
        _enc2_body,
        grid=(grid,),
        in_specs=[
            pl.BlockSpec((2, _ROWS, _D), lambda i: (0, i, 0)),
            pl.BlockSpec((_ROWS, _D), lambda i: (i, 0)),
            pl.BlockSpec((_ROWS, _D), lambda i: (i, 0)),
            pl.BlockSpec((1, _D), lambda i: (0, 0)),
            pl.BlockSpec((_D, _D), lambda i: (0, 0)),
        ],
        out_specs=pl.BlockSpec((_ROWS, _D), lambda i: (i, 0)),
        out_shape=jax.ShapeDtypeStruct((_N, _D), jnp.float32),
    )(s1, h1p, dinv, b1, w2)


def _final_body(s2_ref, h2p_ref, dinv_ref, b2_ref, z_ref):
    z_ref[...] = (s2_ref[0] + s2_ref[1] + h2p_ref[...]) * dinv_ref[...] + b2_ref[...]


def _final(s2, h2p, dinv, b2):
    grid = _N // _ROWS
    return pl.pallas_call(
        _final_body,
        grid=(grid,),
        in_specs=[
            pl.BlockSpec((2, _ROWS, _D), lambda i: (0, i, 0)),
            pl.BlockSpec((_ROWS, _D), lambda i: (i, 0)),
            pl.BlockSpec((_ROWS, _D), lambda i: (i, 0)),
            pl.BlockSpec((1, _D), lambda i: (0, 0)),
        ],
        out_specs=pl.BlockSpec((_ROWS, _D), lambda i: (i, 0)),
        out_shape=jax.ShapeDtypeStruct((_N, _D), jnp.float32),
    )(s2, h2p, dinv, b2)


@functools.partial(
    pl.kernel,
    mesh=plsc.VectorSubcoreMesh(**_SC_MESH),
    compiler_params=pltpu.CompilerParams(needs_layout_passes=False, use_tc_tiling_on_sc=False),
    out_type=jax.ShapeDtypeStruct((2, _NP, 8), jnp.float32),
    scratch_types=[
        pltpu.VMEM((_CH,), jnp.int32),
        pltpu.VMEM((_CH, 8), jnp.float32),
        pltpu.VMEM_SHARED((_NP, 8), jnp.float32),
    ],
)
def _deg_sc(dst_hbm, ones_hbm, zeros_hbm, out_hbm, idx_v, ones_v, deg_sh):
    c = lax.axis_index("c")
    s = lax.axis_index("s")
    wid = c * 16 + s
    base = wid * _EPT
    pltpu.sync_copy(ones_hbm, ones_v)
    pltpu.sync_copy(zeros_hbm, deg_sh.at[pl.ds(s * _RPT, _RPT)])
    plsc.subcore_barrier()

    def body(i, carry):
        off = base + i * _CH
        pltpu.sync_copy(dst_hbm.at[pl.ds(off, _CH)], idx_v)
        pltpu.sync_copy(ones_v, deg_sh.at[idx_v], add=True)
        return carry

    lax.fori_loop(0, _NCH, body, 0)
    plsc.subcore_barrier()
    pltpu.sync_copy(deg_sh.at[pl.ds(s * _RPT, _RPT)],
                    out_hbm.at[c, pl.ds(s * _RPT, _RPT)])


@functools.partial(
    pl.kernel,
    mesh=plsc.VectorSubcoreMesh(**_SC_MESH),
    compiler_params=pltpu.CompilerParams(needs_layout_passes=False, use_tc_tiling_on_sc=False),
    out_type=jax.ShapeDtypeStruct((2, _NP, _D), jnp.float32),
    scratch_types=[
        pltpu.VMEM((_CH,), jnp.int32),
        pltpu.VMEM((_CH,), jnp.int32),
        pltpu.VMEM((_CH, _D), jnp.float32),
        pltpu.VMEM_SHARED((_NP, _D), jnp.float32),
        pltpu.SemaphoreType.DMA,
    ],
)
def _scatter_sc(src_hbm, dst_hbm, h_hbm, zeros_hbm, out_hbm, src_v, dst_v, rows_v,
                acc_sh, sem):
    c = lax.axis_index("c")
    s = lax.axis_index("s")
    wid = c * 16 + s
    base = wid * _EPT
    pltpu.sync_copy(zeros_hbm, acc_sh.at[pl.ds(s * _RPT, _RPT)])
    plsc.subcore_barrier()

    def body(i, carry):
        off = base + i * _CH
        pltpu.sync_copy(src_hbm.at[pl.ds(off, _CH)], src_v)
        pltpu.sync_copy(dst_hbm.at[pl.ds(off, _CH)], dst_v)
        pltpu.async_copy(h_hbm.at[src_v], rows_v, sem).wait()
        pltpu.sync_copy(rows_v, acc_sh.at[dst_v], add=True)
        return carry

    lax.fori_loop(0, _NCH, body, 0)
    plsc.subcore_barrier()
    pltpu.sync_copy(acc_sh.at[pl.ds(s * _RPT, _RPT)],
                    out_hbm.at[c, pl.ds(s * _RPT, _RPT)])


@functools.partial(
    pl.kernel,
    mesh=plsc.VectorSubcoreMesh(**_SC_MESH),
    compiler_params=pltpu.CompilerParams(needs_layout_passes=False, use_tc_tiling_on_sc=False),
    out_type=jax.ShapeDtypeStruct((_E,), jnp.float32),
    scratch_types=[
        pltpu.VMEM((_CH,), jnp.int32),
        pltpu.VMEM((_CH,), jnp.int32),
        pltpu.VMEM((_CH, _D), jnp.float32),
        pltpu.VMEM((_CH, _D), jnp.float32),
        pltpu.VMEM((_EPT,), jnp.float32),
        pltpu.SemaphoreType.DMA,
        pltpu.SemaphoreType.DMA,
    ],
)
def _decode_sc(u_hbm, v_hbm, z_hbm, out_hbm, iu_v, iv_v, zu_v, zv_v,
               acc_v, semu, semv):
    c = lax.axis_index("c")
    s = lax.axis_index("s")
    wid = c * 16 + s
    base = wid * _EPT

    def chunk(i, carry):
        off = base + i * _CH
        pltpu.sync_copy(u_hbm.at[pl.ds(off, _CH)], iu_v)
        pltpu.sync_copy(v_hbm.at[pl.ds(off, _CH)], iv_v)
        cu = pltpu.async_copy(z_hbm.at[iu_v], zu_v, semu)
        cv = pltpu.async_copy(z_hbm.at[iv_v], zv_v, semv)
        cu.wait()
        cv.wait()

        def group(g, carry2):
            e0 = g * 16
            lane = lax.iota(jnp.int32, 16)
            vec = jnp.zeros((16,), jnp.float32)
            for e in range(16):
                acc = zu_v[e0 + e, pl.ds(0, 16)] * zv_v[e0 + e, pl.ds(0, 16)]
                for j in range(1, _D // 16):
                    acc += (zu_v[e0 + e, pl.ds(j * 16, 16)]
                            * zv_v[e0 + e, pl.ds(j * 16, 16)])
                vec = jnp.where(lane == e, jnp.sum(acc), vec)
            acc_v[pl.ds(i * _CH + e0, 16)] = vec
            return carry2

        lax.fori_loop(0, _CH // 16, group, 0)
        return carry

    lax.fori_loop(0, _NCH, chunk, 0)
    pltpu.sync_copy(acc_v, out_hbm.at[pl.ds(base, _EPT)])


_USE_SC_DEG = True
_USE_SC_SCATTER = True
_USE_SC_DECODE = True


def _deg_parts(dst):
    if not _USE_SC_DEG:
        half = _E // 2
        d0 = jnp.zeros((_N,), jnp.float32).at[dst[:half]].add(1.0)
        d1 = jnp.zeros((_N,), jnp.float32).at[dst[half:]].add(1.0)
        return jnp.stack([d0, d1])
    ones = jnp.ones((_CH, 8), jnp.float32)
    zeros = jnp.zeros((_RPT, 8), jnp.float32)
    parts = _deg_sc(dst, ones, zeros)
    return parts[:, :_N, 0]


def _scatter_parts(src, dst, h):
    if not _USE_SC_SCATTER:
        half = _E // 2
        s0 = jnp.zeros((_N, _D), jnp.float32).at[dst[:half]].add(h[src[:half]])
        s1 = jnp.zeros((_N, _D), jnp.float32).at[dst[half:]].add(h[src[half:]])
        return jnp.stack([s0, s1])
    zeros = jnp.zeros((_RPT, _D), jnp.float32)
    parts = _scatter_sc(src, dst, h, zeros)
    return parts[:, :_N, :]


def _decode(u, v, z):
    if not _USE_SC_DECODE:
        return (z[u] * z[v]).sum(axis=-1)
    return _decode_sc(u, v, z)


def kernel(x, edge_index, pos_edge_index, neg_edge_index, W1, b1, W2, b2):
    src = edge_index[0]
    dst = edge_index[1]
    u = jnp.concatenate([pos_edge_index[0], neg_edge_index[0]])
    v = jnp.concatenate([pos_edge_index[1], neg_edge_index[1]])
    deg_parts = _deg_parts(dst)
    dinv = _prep(deg_parts)
    h1p = _enc1(x, W1, dinv)
    s1 = _scatter_parts(src, dst, h1p)
    h2p = _enc2(s1, h1p, dinv, b1.reshape(1, _D), W2)
    s2 = _scatter_parts(src, dst, h2p)
    z = _final(s2, h2p, dinv, b2.reshape(1, _D))
    return _decode(u, v, z)
